# Initial kernel scaffold; baseline (speedup 1.0000x reference)
import jax
import jax.numpy as jnp
from jax.experimental import pallas as pl


def kernel(op_features, edge_index, batch_index, W, a_src, a_dst, bias, W1, b1, W2, b2):
    raise NotImplementedError("write your pallas kernel here")


# speedup vs baseline: 50.3828x; 50.3828x over previous
import functools

import jax
import jax.numpy as jnp
from jax import lax
from jax.experimental import pallas as pl
from jax.experimental.pallas import tpu as pltpu
from jax.experimental.pallas import tpu_sc as plsc

N = 10000
IN_DIM = 128
H = 4
C = 16
OUT = H * C
G = 64
HID = 128

NC, NS = 2, 16
N_PAD = 10240
E = 320000
E_PAD = 327680
EDGES_PER_TILE = E_PAD // (NC * NS)
CHUNK = 128
CHUNKS = EDGES_PER_TILE // CHUNK
GRP = 8
NGRP = CHUNKS // GRP
ROWS_PER_TILE = N_PAD // NS

ROW_BLK = 512
POOL_BLK = 400


def _prep_body(x_ref, w_ref, as_ref, ad_ref, t_ref, dexp_ref, init_ref):
    h = jnp.dot(x_ref[...], w_ref[...], preferred_element_type=jnp.float32)
    s_exp = jnp.dot(h, as_ref[...], preferred_element_type=jnp.float32)
    d_exp = jnp.dot(h, ad_ref[...], preferred_element_type=jnp.float32)
    e = s_exp + d_exp
    w_self = jnp.exp(jnp.where(e >= 0.0, e, 0.2 * e))
    t_ref[:, :OUT] = h
    t_ref[:, OUT:] = s_exp
    dexp_ref[:, :OUT] = d_exp
    dexp_ref[:, OUT:] = jnp.zeros_like(d_exp)
    init_ref[:, :OUT] = 0.5 * w_self * h
    init_ref[:, OUT:] = 0.5 * w_self


_prep_call = pl.pallas_call(
    _prep_body,
    grid=(N_PAD // ROW_BLK,),
    in_specs=[
        pl.BlockSpec((ROW_BLK, IN_DIM), lambda i: (i, 0)),
        pl.BlockSpec((IN_DIM, OUT), lambda i: (0, 0)),
        pl.BlockSpec((OUT, OUT), lambda i: (0, 0)),
        pl.BlockSpec((OUT, OUT), lambda i: (0, 0)),
    ],
    out_specs=[
        pl.BlockSpec((ROW_BLK, 2 * OUT), lambda i: (i, 0)),
        pl.BlockSpec((ROW_BLK, 2 * OUT), lambda i: (i, 0)),
        pl.BlockSpec((ROW_BLK, 2 * OUT), lambda i: (i, 0)),
    ],
    out_shape=[
        jax.ShapeDtypeStruct((N_PAD, 2 * OUT), jnp.float32),
        jax.ShapeDtypeStruct((N_PAD, 2 * OUT), jnp.float32),
        jax.ShapeDtypeStruct((N_PAD, 2 * OUT), jnp.float32),
    ],
)


@functools.partial(
    pl.kernel,
    out_type=jax.ShapeDtypeStruct((NC, N_PAD, 2 * OUT), jnp.float32),
    mesh=plsc.VectorSubcoreMesh(core_axis_name="c", subcore_axis_name="s"),
    scratch_types=[
        pltpu.VMEM((GRP, CHUNK), jnp.int32),
        pltpu.VMEM((GRP, CHUNK), jnp.int32),
        pltpu.VMEM((CHUNK, 2 * OUT), jnp.float32),
        pltpu.VMEM((CHUNK, 2 * OUT), jnp.float32),
        pltpu.VMEM_SHARED((N_PAD, 2 * OUT), jnp.float32),
        pltpu.SemaphoreType.DMA,
        pltpu.SemaphoreType.DMA,
    ],
)
def _edge_kernel(t_hbm, dexp_hbm, src_hbm, dst_hbm, init_hbm, out_hbm,
                 src_v, dst_v, rows_v, drows_v, acc_sh, sem1, sem2):
    c = lax.axis_index("c")
    s = lax.axis_index("s")
    wid = s * NC + c
    row0 = s * ROWS_PER_TILE

    pltpu.sync_copy(init_hbm.at[pl.ds(row0, ROWS_PER_TILE)],
                    acc_sh.at[pl.ds(row0, ROWS_PER_TILE)])

    chunk0 = wid * CHUNKS
    plsc.subcore_barrier()

    def grp_body(g, carry0):
        pltpu.sync_copy(src_hbm.at[pl.ds(chunk0 + g * GRP, GRP)], src_v)
        pltpu.sync_copy(dst_hbm.at[pl.ds(chunk0 + g * GRP, GRP)], dst_v)

        def chunk_body(j, carry):
            g1 = pltpu.async_copy(t_hbm.at[src_v.at[j]], rows_v, sem1)
            g2 = pltpu.async_copy(dexp_hbm.at[dst_v.at[j]], drows_v, sem2)
            g1.wait()
            g2.wait()

            def edge_body(k, carry2):
                for q in range(H):
                    sv = rows_v[k, pl.ds(OUT + q * 16, 16)]
                    dv = drows_v[k, pl.ds(q * 16, 16)]
                    e = sv + dv
                    w = jnp.exp(jnp.where(e >= 0.0, e, 0.2 * e))
                    rows_v[k, pl.ds(OUT + q * 16, 16)] = w
                    hv = rows_v[k, pl.ds(q * 16, 16)]
                    rows_v[k, pl.ds(q * 16, 16)] = hv * w
                return carry2

            lax.fori_loop(0, CHUNK, edge_body, 0)
            pltpu.sync_copy(rows_v, acc_sh.at[dst_v.at[j]], add=True)
            return carry

        lax.fori_loop(0, GRP, chunk_body, 0)
        return carry0

    lax.fori_loop(0, NGRP, grp_body, 0)
    plsc.subcore_barrier()
    pltpu.sync_copy(acc_sh.at[pl.ds(row0, ROWS_PER_TILE)],
                    out_hbm.at[c, pl.ds(row0, ROWS_PER_TILE)])


def _pool_body(acc_ref, bias_ref, b_ref, emb_ref, sums_ref, cnt_ref):
    i = pl.program_id(0)
    acc = acc_ref[0] + acc_ref[1]
    gat = acc[:, :OUT] / acc[:, OUT:] + bias_ref[...]
    emb = jnp.where(gat > 0.0, gat, jnp.exp(jnp.minimum(gat, 0.0)) - 1.0)
    emb_ref[...] = emb
    iota = lax.broadcasted_iota(jnp.int32, (1, G), 1)
    onehot = (b_ref[...] == iota).astype(jnp.float32)
    psums = lax.dot_general(onehot, emb, (((0,), (0,)), ((), ())),
                            preferred_element_type=jnp.float32)
    pcnt = lax.dot_general(onehot, jnp.ones((POOL_BLK, 1), jnp.float32),
                           (((0,), (0,)), ((), ())),
                           preferred_element_type=jnp.float32)

    @pl.when(i == 0)
    def _init():
        sums_ref[...] = psums
        cnt_ref[...] = pcnt

    @pl.when(i > 0)
    def _accum():
        sums_ref[...] += psums
        cnt_ref[...] += pcnt


_pool_call = pl.pallas_call(
    _pool_body,
    grid=(N // POOL_BLK,),
    in_specs=[
        pl.BlockSpec((NC, POOL_BLK, 2 * OUT), lambda i: (0, i, 0)),
        pl.BlockSpec((1, OUT), lambda i: (0, 0)),
        pl.BlockSpec((POOL_BLK, 1), lambda i: (i, 0)),
    ],
    out_specs=[
        pl.BlockSpec((POOL_BLK, OUT), lambda i: (i, 0)),
        pl.BlockSpec((G, OUT), lambda i: (0, 0)),
        pl.BlockSpec((G, 1), lambda i: (0, 0)),
    ],
    out_shape=[
        jax.ShapeDtypeStruct((N, OUT), jnp.float32),
        jax.ShapeDtypeStruct((G, OUT), jnp.float32),
        jax.ShapeDtypeStruct((G, 1), jnp.float32),
    ],
)


def _dec_body(emb_ref, b_ref, sums_ref, cnt_ref, w1a_ref, w1b_ref, b1_ref,
              w2_ref, b2_ref, scores_ref, ge_ref):
    ge = sums_ref[...] / jnp.maximum(cnt_ref[...], 1.0)

    @pl.when(pl.program_id(0) == 0)
    def _write_ge():
        ge_ref[...] = ge

    iota = lax.broadcasted_iota(jnp.int32, (1, G), 1)
    onehot = (b_ref[...] == iota).astype(jnp.float32)
    gexp = jnp.dot(onehot, ge, preferred_element_type=jnp.float32)
    hid = jnp.dot(emb_ref[...], w1a_ref[...], preferred_element_type=jnp.float32)
    hid += jnp.dot(gexp, w1b_ref[...], preferred_element_type=jnp.float32)
    hid = jnp.maximum(hid + b1_ref[...], 0.0)
    scores_ref[...] = jnp.dot(hid, w2_ref[...],
                              preferred_element_type=jnp.float32) + b2_ref[...]


_dec_call = pl.pallas_call(
    _dec_body,
    grid=(N // POOL_BLK,),
    in_specs=[
        pl.BlockSpec((POOL_BLK, OUT), lambda i: (i, 0)),
        pl.BlockSpec((POOL_BLK, 1), lambda i: (i, 0)),
        pl.BlockSpec((G, OUT), lambda i: (0, 0)),
        pl.BlockSpec((G, 1), lambda i: (0, 0)),
        pl.BlockSpec((OUT, HID), lambda i: (0, 0)),
        pl.BlockSpec((G, HID), lambda i: (0, 0)),
        pl.BlockSpec((1, HID), lambda i: (0, 0)),
        pl.BlockSpec((HID, 1), lambda i: (0, 0)),
        pl.BlockSpec((1, 1), lambda i: (0, 0)),
    ],
    out_specs=[
        pl.BlockSpec((POOL_BLK, 1), lambda i: (i, 0)),
        pl.BlockSpec((G, OUT), lambda i: (0, 0)),
    ],
    out_shape=[
        jax.ShapeDtypeStruct((N, 1), jnp.float32),
        jax.ShapeDtypeStruct((G, OUT), jnp.float32),
    ],
)


def kernel(op_features, edge_index, batch_index, W, a_src, a_dst, bias,
           W1, b1, W2, b2):
    x = jnp.pad(op_features, ((0, N_PAD - N), (0, 0)))
    blk_mask = jnp.kron(jnp.eye(H, dtype=jnp.float32),
                        jnp.ones((C, C), jnp.float32))
    A_s = a_src.reshape(OUT, 1) * blk_mask
    A_d = a_dst.reshape(OUT, 1) * blk_mask
    pad_idx = jnp.full((E_PAD - E,), N, jnp.int32)
    src2d = jnp.concatenate([edge_index[0], pad_idx]).reshape(E_PAD // CHUNK,
                                                              CHUNK)
    dst2d = jnp.concatenate([edge_index[1], pad_idx]).reshape(E_PAD // CHUNK,
                                                              CHUNK)
    b_col = batch_index.reshape(N, 1)

    t_tab, dexp_tab, init_half = _prep_call(x, W, A_s, A_d)

    acc2 = _edge_kernel(t_tab, dexp_tab, src2d, dst2d, init_half)
    acc2 = acc2[:, :N, :]

    emb, sums, cnt = _pool_call(acc2, bias.reshape(1, OUT), b_col)
    scores, ge = _dec_call(emb, b_col, sums, cnt, W1[:OUT], W1[OUT:],
                           b1.reshape(1, HID), W2, b2.reshape(1, 1))
    return scores.reshape(N), ge
